# Initial kernel scaffold; baseline (speedup 1.0000x reference)
import jax
import jax.numpy as jnp
from jax.experimental import pallas as pl


def kernel(X_oh, h_pos, J):
    raise NotImplementedError("write your pallas kernel here")


# speedup vs baseline: 10.8781x; 10.8781x over previous
import functools

import jax
import jax.numpy as jnp
from jax.experimental import pallas as pl


def _matmul_body(x_ref, w_ref, h_ref, o_ref, *, Q, TN):
    t = pl.program_id(0)
    col0 = t * TN
    row_j = jax.lax.broadcasted_iota(jnp.int32, (w_ref.shape[0], 1), 0) // Q
    col_i = (jax.lax.broadcasted_iota(jnp.int32, (1, TN), 1) + col0) // Q
    mask = row_j < col_i
    wm = jnp.where(mask, w_ref[...], 0.0)
    acc = jnp.dot(x_ref[...], wm, preferred_element_type=jnp.float32)
    o_ref[...] = acc + h_ref[...]


def kernel(X_oh, h_pos, J):
    M, L, Q = X_oh.shape
    LQ = L * Q
    W = jnp.transpose(J, (1, 2, 0, 3)).reshape(LQ, LQ)
    Xf = X_oh.reshape(M, LQ)
    hf = h_pos.reshape(1, LQ)

    TN = 128
    n_col = LQ // TN

    out = pl.pallas_call(
        functools.partial(_matmul_body, Q=Q, TN=TN),
        grid=(n_col,),
        in_specs=[
            pl.BlockSpec((M, LQ), lambda t: (0, 0)),
            pl.BlockSpec((LQ, TN), lambda t: (0, t)),
            pl.BlockSpec((1, TN), lambda t: (0, t)),
        ],
        out_specs=pl.BlockSpec((M, TN), lambda t: (0, t)),
        out_shape=jax.ShapeDtypeStruct((M, LQ), jnp.float32),
    )(Xf, W, hf)
    return out.reshape(M, L, Q)


# speedup vs baseline: 12.8283x; 1.1793x over previous
import functools

import jax
import jax.numpy as jnp
from jax.experimental import pallas as pl


def _matmul_body(x_ref, w_ref, h_ref, o_ref, *, Q, TN):
    t = pl.program_id(0)
    col0 = t * TN
    row_j = jax.lax.broadcasted_iota(jnp.int32, (w_ref.shape[0], 1), 0) // Q
    col_i = (jax.lax.broadcasted_iota(jnp.int32, (1, TN), 1) + col0) // Q
    mask = row_j < col_i
    wm = jnp.where(mask, w_ref[...], jnp.zeros((), w_ref.dtype))
    acc = jnp.dot(x_ref[...], wm, preferred_element_type=jnp.float32)
    o_ref[...] = acc + h_ref[...]


def kernel(X_oh, h_pos, J):
    M, L, Q = X_oh.shape
    LQ = L * Q
    W = jnp.transpose(J.astype(jnp.bfloat16), (1, 2, 0, 3)).reshape(LQ, LQ)
    Xf = X_oh.reshape(M, LQ).astype(jnp.bfloat16)
    hf = h_pos.reshape(1, LQ)

    TN = 128
    n_col = LQ // TN

    out = pl.pallas_call(
        functools.partial(_matmul_body, Q=Q, TN=TN),
        grid=(n_col,),
        in_specs=[
            pl.BlockSpec((M, LQ), lambda t: (0, 0)),
            pl.BlockSpec((LQ, TN), lambda t: (0, t)),
            pl.BlockSpec((1, TN), lambda t: (0, t)),
        ],
        out_specs=pl.BlockSpec((M, TN), lambda t: (0, t)),
        out_shape=jax.ShapeDtypeStruct((M, LQ), jnp.float32),
    )(Xf, W, hf)
    return out.reshape(M, L, Q)
